# baseline (speedup 1.0000x reference)
import functools

import jax
import jax.numpy as jnp
from jax import lax
from jax.experimental import pallas as pl
from jax.experimental.pallas import tpu as pltpu
from jax.experimental.pallas import tpu_sc as plsc

_GROUP = 128
_GROUPS_PER_CHUNK = 5
_EMBP = 64
_PAD = 128


@functools.lru_cache(maxsize=None)
def _make_gather(n_rows: int, vocab: int):
    info = plsc.get_sparse_core_info()
    nw = info.num_cores * info.num_subcores
    chunk_rows = _GROUP * _GROUPS_PER_CHUNK
    assert n_rows % (nw * chunk_rows * 2) == 0
    chunks_per_w = n_rows // (nw * chunk_rows)

    mesh = plsc.VectorSubcoreMesh(core_axis_name="c", subcore_axis_name="s")

    @functools.partial(
        pl.kernel,
        mesh=mesh,
        out_type=jax.ShapeDtypeStruct((n_rows, _PAD), jnp.float32),
        scratch_types=[
            pltpu.VMEM((2, _GROUPS_PER_CHUNK, _GROUP), jnp.int32),
            pltpu.VMEM((2, chunk_rows, _EMBP), jnp.float32),
            pltpu.SemaphoreType.DMA,
            pltpu.SemaphoreType.DMA,
            pltpu.SemaphoreType.DMA,
        ],
        compiler_params=pltpu.CompilerParams(use_tc_tiling_on_sc=False),
    )
    def gather_k(table_hbm, idx_hbm, out_hbm, idx_v, rows_v, gsem, osem0, osem1):
        cid = lax.axis_index("c")
        sid = lax.axis_index("s")
        wid = sid * info.num_cores + cid
        c0 = wid * chunks_per_w
        osems = (osem0, osem1)

        def out_dst(c):
            return out_hbm.at[pl.ds((c0 + c) * chunk_rows, chunk_rows),
                              pl.ds(0, _EMBP)]

        def do_chunk(c, buf, first):
            pltpu.sync_copy(idx_hbm.at[c0 + c], idx_v.at[buf])
            @pl.when(jnp.logical_not(first))
            def _():
                pltpu.make_async_copy(rows_v.at[buf], out_dst(c), osems[buf]).wait()

            handles = []
            for j in range(_GROUPS_PER_CHUNK):
                handles.append(
                    pltpu.async_copy(
                        table_hbm.at[idx_v.at[buf].at[j]],
                        rows_v.at[buf].at[pl.ds(j * _GROUP, _GROUP)],
                        gsem,
                    )
                )
            for h in handles:
                h.wait()
            pltpu.make_async_copy(rows_v.at[buf], out_dst(c), osems[buf]).start()

        def pair_body(p, carry):
            do_chunk(2 * p, 0, p == 0)
            do_chunk(2 * p + 1, 1, p == 0)
            return carry

        lax.fori_loop(0, chunks_per_w // 2, pair_body, 0)
        pltpu.make_async_copy(
            rows_v.at[0], out_dst(chunks_per_w - 2), osem0).wait()
        pltpu.make_async_copy(
            rows_v.at[1], out_dst(chunks_per_w - 1), osem1).wait()

    return gather_k


def _matmul_kernel(a_ref, w_ref, b_ref, o_ref):
    a = a_ref[...]
    lane = lax.broadcasted_iota(jnp.int32, a.shape, 1) % _PAD
    a = jnp.where(lane < _EMBP, a, 0.0)
    acc = lax.dot_general(
        a, w_ref[...],
        dimension_numbers=(((1,), (1,)), ((), ())),
        preferred_element_type=jnp.float32,
    )
    o_ref[...] = acc + b_ref[...]


def _tc_matmul(flat, Wp, b2):
    batch, kp = flat.shape
    out_dim = Wp.shape[0]
    bm = 256
    return pl.pallas_call(
        _matmul_kernel,
        grid=(batch // bm,),
        in_specs=[
            pl.BlockSpec((bm, kp), lambda i: (i, 0)),
            pl.BlockSpec((out_dim, kp), lambda i: (0, 0)),
            pl.BlockSpec((1, out_dim), lambda i: (0, 0)),
        ],
        out_specs=pl.BlockSpec((bm, out_dim), lambda i: (i, 0)),
        out_shape=jax.ShapeDtypeStruct((batch, out_dim), jnp.float32),
    )(flat, Wp, b2)


def kernel(x, table, W, b):
    batch, enc = x.shape
    vocab, emb = table.shape
    out_dim = W.shape[0]
    n_rows = batch * enc
    chunk_rows = _GROUP * _GROUPS_PER_CHUNK
    idx = x.reshape(n_rows // chunk_rows, _GROUPS_PER_CHUNK, _GROUP).astype(jnp.int32)

    z = jnp.zeros((vocab // 2, _EMBP - emb), table.dtype)
    table_p = jnp.concatenate(
        [table[0::2], z, table[1::2], z], axis=1
    )
    gathered = _make_gather(n_rows, vocab)(
        table_p.reshape(vocab, _EMBP), idx
    )
    flat = gathered.reshape(batch, enc * _PAD)

    Wp = jnp.pad(
        W.reshape(out_dim, enc, emb), ((0, 0), (0, 0), (0, _PAD - emb))
    ).reshape(out_dim, enc * _PAD)

    return _tc_matmul(flat, Wp, b.reshape(1, out_dim))


# speedup vs baseline: 3.2500x; 3.2500x over previous
import functools

import jax
import jax.numpy as jnp
from jax import lax
from jax.experimental import pallas as pl
from jax.experimental.pallas import tpu as pltpu
from jax.experimental.pallas import tpu_sc as plsc

_GROUP = 128
_GROUPS_PER_CHUNK = 5
_EMBP = 64
_PAD = 128


@functools.lru_cache(maxsize=None)
def _make_gather(n_rows: int, vocab: int):
    info = plsc.get_sparse_core_info()
    nw = info.num_cores * info.num_subcores
    chunk_rows = _GROUP * _GROUPS_PER_CHUNK
    assert n_rows % (nw * chunk_rows * 2) == 0
    chunks_per_w = n_rows // (nw * chunk_rows)

    mesh = plsc.VectorSubcoreMesh(core_axis_name="c", subcore_axis_name="s")

    @functools.partial(
        pl.kernel,
        mesh=mesh,
        out_type=jax.ShapeDtypeStruct((n_rows, _PAD), jnp.float32),
        scratch_types=[
            pltpu.VMEM((2, _GROUPS_PER_CHUNK, _GROUP), jnp.int32),
            pltpu.VMEM((2, chunk_rows, _EMBP), jnp.float32),
            pltpu.SemaphoreType.DMA,
            pltpu.SemaphoreType.DMA,
            pltpu.SemaphoreType.DMA,
        ],
        compiler_params=pltpu.CompilerParams(use_tc_tiling_on_sc=False),
    )
    def gather_k(table_hbm, idx_hbm, out_hbm, idx_v, rows_v, gsem, osem0, osem1):
        cid = lax.axis_index("c")
        sid = lax.axis_index("s")
        wid = sid * info.num_cores + cid
        c0 = wid * chunks_per_w
        osems = (osem0, osem1)

        def out_dst(c):
            return out_hbm.at[pl.ds((c0 + c) * chunk_rows, chunk_rows),
                              pl.ds(0, _EMBP)]

        def do_chunk(c, buf, first):
            pltpu.sync_copy(idx_hbm.at[c0 + c], idx_v.at[buf])
            @pl.when(jnp.logical_not(first))
            def _():
                pltpu.make_async_copy(rows_v.at[buf], out_dst(c), osems[buf]).wait()

            handles = []
            for j in range(_GROUPS_PER_CHUNK):
                handles.append(
                    pltpu.async_copy(
                        table_hbm.at[idx_v.at[buf].at[j]],
                        rows_v.at[buf].at[pl.ds(j * _GROUP, _GROUP)],
                        gsem,
                    )
                )
            for h in handles:
                h.wait()
            pltpu.make_async_copy(rows_v.at[buf], out_dst(c), osems[buf]).start()

        def pair_body(p, carry):
            do_chunk(2 * p, 0, p == 0)
            do_chunk(2 * p + 1, 1, p == 0)
            return carry

        lax.fori_loop(0, chunks_per_w // 2, pair_body, 0)
        pltpu.make_async_copy(
            rows_v.at[0], out_dst(chunks_per_w - 2), osem0).wait()
        pltpu.make_async_copy(
            rows_v.at[1], out_dst(chunks_per_w - 1), osem1).wait()

    return gather_k


def _matmul_kernel(a_ref, w_ref, b_ref, o_ref):
    a = a_ref[...]
    lane = lax.broadcasted_iota(jnp.int32, a.shape, 1) % _PAD
    a = jnp.where(lane < _EMBP, a, 0.0)
    acc = lax.dot_general(
        a, w_ref[...],
        dimension_numbers=(((1,), (1,)), ((), ())),
        preferred_element_type=jnp.float32,
    )
    o_ref[...] = acc + b_ref[...]


def _tc_matmul(flat, Wp, b2):
    batch, kp = flat.shape
    out_dim = Wp.shape[0]
    bm = 256
    return pl.pallas_call(
        _matmul_kernel,
        grid=(batch // bm,),
        in_specs=[
            pl.BlockSpec((bm, kp), lambda i: (i, 0)),
            pl.BlockSpec((out_dim, kp), lambda i: (0, 0)),
            pl.BlockSpec((1, out_dim), lambda i: (0, 0)),
        ],
        out_specs=pl.BlockSpec((bm, out_dim), lambda i: (i, 0)),
        out_shape=jax.ShapeDtypeStruct((batch, out_dim), jnp.float32),
    )(flat, Wp, b2)


def kernel(x, table, W, b):
    batch, enc = x.shape
    vocab, emb = table.shape
    out_dim = W.shape[0]
    n_rows = batch * enc
    chunk_rows = _GROUP * _GROUPS_PER_CHUNK
    idx = x.reshape(n_rows // chunk_rows, _GROUPS_PER_CHUNK, _GROUP).astype(jnp.int32)

    table_p = jnp.pad(table, ((0, 0), (0, _EMBP - emb))).reshape(
        vocab // 2, 2 * _EMBP
    )
    gathered = _make_gather(n_rows, vocab)(
        table_p.reshape(vocab, _EMBP), idx
    )
    flat = gathered.reshape(batch, enc * _PAD)

    Wp = jnp.pad(
        W.reshape(out_dim, enc, emb), ((0, 0), (0, 0), (0, _PAD - emb))
    ).reshape(out_dim, enc * _PAD)

    return _tc_matmul(flat, Wp, b.reshape(1, out_dim))


# speedup vs baseline: 3.6026x; 1.1085x over previous
import functools

import jax
import jax.numpy as jnp
from jax import lax
from jax.experimental import pallas as pl
from jax.experimental.pallas import tpu as pltpu
from jax.experimental.pallas import tpu_sc as plsc

_GROUP = 128
_GROUPS_PER_CHUNK = 10
_EMBP = 64


@functools.lru_cache(maxsize=None)
def _make_gather(n_rows: int, vocab: int):
    info = plsc.get_sparse_core_info()
    nw = info.num_cores * info.num_subcores
    chunk_rows = _GROUP * _GROUPS_PER_CHUNK
    half_rows = chunk_rows // 2
    assert n_rows % (nw * chunk_rows) == 0
    chunks_per_w = n_rows // (nw * chunk_rows)

    mesh = plsc.VectorSubcoreMesh(core_axis_name="c", subcore_axis_name="s")

    @functools.partial(
        pl.kernel,
        mesh=mesh,
        out_type=jax.ShapeDtypeStruct((n_rows // 2, 2 * _EMBP), jnp.float32),
        scratch_types=[
            pltpu.VMEM((_GROUPS_PER_CHUNK, _GROUP), jnp.int32),
            pltpu.VMEM((chunk_rows, _EMBP), jnp.float32),
            pltpu.SemaphoreType.DMA,
            pltpu.SemaphoreType.DMA,
        ],
        compiler_params=pltpu.CompilerParams(use_tc_tiling_on_sc=False),
    )
    def gather_k(table_hbm, idx_hbm, out_hbm, idx_v, rows_v, gsem, osem):
        cid = lax.axis_index("c")
        sid = lax.axis_index("s")
        wid = sid * info.num_cores + cid
        c0 = wid * chunks_per_w

        def chunk_body(c, carry):
            pltpu.sync_copy(idx_hbm.at[c0 + c], idx_v)
            handles = []
            for j in range(_GROUPS_PER_CHUNK):
                handles.append(
                    pltpu.async_copy(
                        table_hbm.at[idx_v.at[j]],
                        rows_v.at[pl.ds(j * _GROUP, _GROUP)],
                        gsem,
                    )
                )
            for h in handles:
                h.wait()
            r0 = (c0 + c) * half_rows
            even = pltpu.make_async_copy(
                rows_v.at[pl.ds(0, half_rows)],
                out_hbm.at[pl.ds(r0, half_rows), pl.ds(0, _EMBP)],
                osem,
            )
            odd = pltpu.make_async_copy(
                rows_v.at[pl.ds(half_rows, half_rows)],
                out_hbm.at[pl.ds(r0, half_rows), pl.ds(_EMBP, _EMBP)],
                osem,
            )
            even.start()
            odd.start()
            even.wait()
            odd.wait()
            return carry

        lax.fori_loop(0, chunks_per_w, chunk_body, 0)

    return gather_k


def _matmul_kernel(a_ref, w_ref, b_ref, o_ref):
    acc = lax.dot_general(
        a_ref[...], w_ref[...],
        dimension_numbers=(((1,), (1,)), ((), ())),
        preferred_element_type=jnp.float32,
    )
    o_ref[...] = acc + b_ref[...]


def _tc_matmul(flat, Wp, b2):
    batch, kp = flat.shape
    out_dim = Wp.shape[0]
    bm = 256
    return pl.pallas_call(
        _matmul_kernel,
        grid=(batch // bm,),
        in_specs=[
            pl.BlockSpec((bm, kp), lambda i: (i, 0)),
            pl.BlockSpec((out_dim, kp), lambda i: (0, 0)),
            pl.BlockSpec((1, out_dim), lambda i: (0, 0)),
        ],
        out_specs=pl.BlockSpec((bm, out_dim), lambda i: (i, 0)),
        out_shape=jax.ShapeDtypeStruct((batch, out_dim), jnp.float32),
    )(flat, Wp, b2)


def kernel(x, table, W, b):
    batch, enc = x.shape
    vocab, emb = table.shape
    out_dim = W.shape[0]
    n_rows = batch * enc
    chunk_rows = _GROUP * _GROUPS_PER_CHUNK
    half_rows = chunk_rows // 2
    n_chunks = n_rows // chunk_rows

    idx = (
        x.reshape(n_chunks, half_rows, 2)
        .transpose(0, 2, 1)
        .reshape(n_chunks, _GROUPS_PER_CHUNK, _GROUP)
        .astype(jnp.int32)
    )

    table_p = jnp.pad(table, ((0, 0), (0, _EMBP - emb))).reshape(
        vocab // 2, 2 * _EMBP
    )
    gathered = _make_gather(n_rows, vocab)(
        table_p.reshape(vocab, _EMBP), idx
    )
    flat = gathered.reshape(batch, enc * _EMBP)

    Wp = jnp.pad(
        W.reshape(out_dim, enc, emb), ((0, 0), (0, 0), (0, _EMBP - emb))
    ).reshape(out_dim, enc * _EMBP)

    return _tc_matmul(flat, Wp, b.reshape(1, out_dim))
